# baseline (speedup 1.0000x reference)
import functools

import jax
import jax.numpy as jnp
from jax import lax
from jax.experimental import pallas as pl
from jax.experimental.pallas import tpu as pltpu
from jax.experimental.pallas import tpu_sc as plsc

VOCAB = 100000
EMBED = 64
BATCH = 1024
CTX = 20

_NC = 2
_NS = 16
_NW = _NC * _NS
_LANES = 16
_BW = BATCH // _NW
_ROWS = _BW * CTX
_IDX_CHUNK = 128
_NCHUNK = _ROWS // _IDX_CHUNK


def _hidden_body(ctx_hbm, table_hbm, out_hbm, idx_v, rows_v, hid_v, sem):
    wid = lax.axis_index("s") * _NC + lax.axis_index("c")
    base = wid * _BW

    pltpu.sync_copy(ctx_hbm.at[wid], idx_v)

    copies = [
        pltpu.async_copy(
            table_hbm.at[idx_v.at[k]],
            rows_v.at[pl.ds(k * _IDX_CHUNK, _IDX_CHUNK)],
            sem,
        )
        for k in range(_NCHUNK)
    ]
    for c in copies:
        c.wait()

    def body(i, _):
        r0 = i * CTX
        for c in range(EMBED // _LANES):
            sl = pl.ds(c * _LANES, _LANES)
            acc = rows_v[r0, sl]
            for t in range(1, CTX):
                acc = acc + rows_v[r0 + t, sl]
            hid_v[i, sl] = acc * (1.0 / CTX)
        return _

    lax.fori_loop(0, _BW, body, None)
    pltpu.sync_copy(hid_v, out_hbm.at[pl.ds(base, _BW)])


@functools.partial(
    pl.kernel,
    mesh=plsc.VectorSubcoreMesh(core_axis_name="c", subcore_axis_name="s"),
    out_type=jax.ShapeDtypeStruct((BATCH, EMBED), jnp.float32),
    scratch_types=[
        pltpu.VMEM((_NCHUNK, _IDX_CHUNK), jnp.int32),
        pltpu.VMEM((_ROWS, EMBED), jnp.float32),
        pltpu.VMEM((_BW, EMBED), jnp.float32),
        pltpu.SemaphoreType.DMA,
    ],
    compiler_params=pltpu.CompilerParams(use_tc_tiling_on_sc=False),
)
def _hidden_sc(ctx_hbm, table_hbm, out_hbm, idx_v, rows_v, hid_v, sem):
    _hidden_body(ctx_hbm, table_hbm, out_hbm, idx_v, rows_v, hid_v, sem)


_VBLK = 25088
_BB = 64


def _proj_body(h_ref, w_ref, b_ref, o_ref):
    o_ref[...] = (
        lax.dot_general(
            h_ref[...],
            w_ref[...],
            (((1,), (1,)), ((), ())),
            preferred_element_type=jnp.float32,
        )
        + b_ref[...]
    )


def _projection(hidden, W, b2d):
    grid = (pl.cdiv(VOCAB, _VBLK), BATCH // _BB)
    return pl.pallas_call(
        _proj_body,
        grid=grid,
        in_specs=[
            pl.BlockSpec((_BB, EMBED), lambda j, i: (i, 0)),
            pl.BlockSpec((_VBLK, EMBED), lambda j, i: (j, 0)),
            pl.BlockSpec((1, _VBLK), lambda j, i: (0, j)),
        ],
        out_specs=pl.BlockSpec((_BB, _VBLK), lambda j, i: (i, j)),
        out_shape=jax.ShapeDtypeStruct((BATCH, VOCAB), jnp.float32),
    )(hidden, W, b2d)


def kernel(context, emb_table, W, b):
    ctx3 = context.reshape(_NW, _NCHUNK, _IDX_CHUNK)
    hidden = _hidden_sc(ctx3, emb_table)
    return _projection(hidden, W, b.reshape(1, VOCAB))


# speedup vs baseline: 1.0327x; 1.0327x over previous
import functools

import jax
import jax.numpy as jnp
from jax import lax
from jax.experimental import pallas as pl
from jax.experimental.pallas import tpu as pltpu
from jax.experimental.pallas import tpu_sc as plsc

VOCAB = 100000
EMBED = 64
BATCH = 1024
CTX = 20

_NC = 2
_NS = 16
_NW = _NC * _NS
_LANES = 16
_BW = BATCH // _NW
_ROWS = _BW * CTX
_IDX_CHUNK = 128
_NCHUNK = _ROWS // _IDX_CHUNK


def _hidden_body(ctx_hbm, table_hbm, out_hbm, idx_v, rows_v, hid_v, sem):
    wid = lax.axis_index("s") * _NC + lax.axis_index("c")
    base = wid * _BW

    pltpu.sync_copy(ctx_hbm.at[pl.ds(base * CTX, _ROWS)], idx_v.at[pl.ds(0, _ROWS)])

    def body(i, _):
        r0 = i * CTX
        ids0 = idx_v[pl.ds(r0, _LANES)]
        ids1 = idx_v[pl.ds(r0 + _LANES, _LANES)]
        copies = []
        for t in range(CTX):
            v = ids0[t] if t < _LANES else ids1[t - _LANES]
            copies.append(
                pltpu.async_copy(
                    table_hbm.at[pl.ds(v, 1)], rows_v.at[pl.ds(t, 1)], sem
                )
            )
        for cp in copies:
            cp.wait()
        for c in range(EMBED // _LANES):
            sl = pl.ds(c * _LANES, _LANES)
            acc = rows_v[0, sl]
            for t in range(1, CTX):
                acc = acc + rows_v[t, sl]
            hid_v[i, sl] = acc * (1.0 / CTX)
        return _

    lax.fori_loop(0, _BW, body, None)
    pltpu.sync_copy(hid_v, out_hbm.at[pl.ds(base, _BW)])


@functools.partial(
    pl.kernel,
    mesh=plsc.VectorSubcoreMesh(core_axis_name="c", subcore_axis_name="s"),
    out_type=jax.ShapeDtypeStruct((BATCH, EMBED), jnp.float32),
    scratch_types=[
        pltpu.VMEM((_ROWS + _LANES,), jnp.int32),
        pltpu.VMEM((CTX, EMBED), jnp.float32),
        pltpu.VMEM((_BW, EMBED), jnp.float32),
        pltpu.SemaphoreType.DMA,
    ],
)
def _hidden_sc(ctx_hbm, table_hbm, out_hbm, idx_v, rows_v, hid_v, sem):
    _hidden_body(ctx_hbm, table_hbm, out_hbm, idx_v, rows_v, hid_v, sem)


_VBLK = 25088
_BB = 64


def _proj_body(h_ref, w_ref, b_ref, o_ref):
    o_ref[...] = (
        lax.dot_general(
            h_ref[...],
            w_ref[...],
            (((1,), (1,)), ((), ())),
            preferred_element_type=jnp.float32,
        )
        + b_ref[...]
    )


def _projection(hidden, W, b2d):
    grid = (pl.cdiv(VOCAB, _VBLK), BATCH // _BB)
    return pl.pallas_call(
        _proj_body,
        grid=grid,
        in_specs=[
            pl.BlockSpec((_BB, EMBED), lambda j, i: (i, 0)),
            pl.BlockSpec((_VBLK, EMBED), lambda j, i: (j, 0)),
            pl.BlockSpec((1, _VBLK), lambda j, i: (0, j)),
        ],
        out_specs=pl.BlockSpec((_BB, _VBLK), lambda j, i: (i, j)),
        out_shape=jax.ShapeDtypeStruct((BATCH, VOCAB), jnp.float32),
    )(hidden, W, b2d)


def kernel(context, emb_table, W, b):
    hidden = _hidden_sc(context.reshape(BATCH * CTX), emb_table)
    return _projection(hidden, W, b.reshape(1, VOCAB))


# speedup vs baseline: 2.2949x; 2.2223x over previous
import functools

import jax
import jax.numpy as jnp
from jax import lax
from jax.experimental import pallas as pl
from jax.experimental.pallas import tpu as pltpu
from jax.experimental.pallas import tpu_sc as plsc

VOCAB = 100000
EMBED = 64
BATCH = 1024
CTX = 20

_NC = 2
_NS = 16
_NW = _NC * _NS
_LANES = 16
_BW = BATCH // _NW
_ROWS = _BW * CTX
_IDX_CHUNK = 128
_NCHUNK = _ROWS // _IDX_CHUNK
_BGROUPS = _BW // _LANES


def _hidden_body(ctx_hbm, tab_hbm, out_hbm, idx_v, val_v, hid_v, sem):
    wid = lax.axis_index("s") * _NC + lax.axis_index("c")
    base = wid * _BW

    pltpu.sync_copy(ctx_hbm.at[wid], idx_v)

    def per_dim(e, _):
        copies = [
            pltpu.async_copy(
                tab_hbm.at[e].at[idx_v.at[k]],
                val_v.at[pl.ds(k * _IDX_CHUNK, _IDX_CHUNK)],
                sem,
            )
            for k in range(_NCHUNK)
        ]
        for cp in copies:
            cp.wait()
        for g in range(_BGROUPS):
            sl0 = g * _LANES
            acc = val_v[pl.ds(sl0, _LANES)]
            for t in range(1, CTX):
                acc = acc + val_v[pl.ds(t * _BW + sl0, _LANES)]
            hid_v[e, pl.ds(sl0, _LANES)] = acc * (1.0 / CTX)
        return _

    lax.fori_loop(0, EMBED, per_dim, None)
    pltpu.sync_copy(hid_v, out_hbm.at[:, pl.ds(base, _BW)])


@functools.partial(
    pl.kernel,
    mesh=plsc.VectorSubcoreMesh(core_axis_name="c", subcore_axis_name="s"),
    out_type=jax.ShapeDtypeStruct((EMBED, BATCH), jnp.float32),
    scratch_types=[
        pltpu.VMEM((_NCHUNK, _IDX_CHUNK), jnp.int32),
        pltpu.VMEM((_ROWS,), jnp.float32),
        pltpu.VMEM((EMBED, _BW), jnp.float32),
        pltpu.SemaphoreType.DMA,
    ],
    compiler_params=pltpu.CompilerParams(use_tc_tiling_on_sc=False),
)
def _hidden_sc(ctx_hbm, tab_hbm, out_hbm, idx_v, val_v, hid_v, sem):
    _hidden_body(ctx_hbm, tab_hbm, out_hbm, idx_v, val_v, hid_v, sem)


_VBLK = 4096


def _proj_body(wt_ref, h_ref, b_ref, o_ref):
    wt_aug = jnp.concatenate([wt_ref[...], b_ref[...]], axis=0)
    h_aug = jnp.concatenate(
        [h_ref[...], jnp.ones((1, BATCH), jnp.float32)], axis=0
    )
    o_ref[...] = lax.dot_general(
        wt_aug,
        h_aug,
        (((0,), (0,)), ((), ())),
        preferred_element_type=jnp.float32,
    )


def _projection(hidden_t, W_t, b2):
    logits_t = pl.pallas_call(
        _proj_body,
        grid=(pl.cdiv(VOCAB, _VBLK),),
        in_specs=[
            pl.BlockSpec((EMBED, _VBLK), lambda j: (0, j)),
            pl.BlockSpec((EMBED, BATCH), lambda j: (0, 0)),
            pl.BlockSpec((1, _VBLK), lambda j: (0, j)),
        ],
        out_specs=pl.BlockSpec((_VBLK, BATCH), lambda j: (j, 0)),
        out_shape=jax.ShapeDtypeStruct((VOCAB, BATCH), jnp.float32),
    )(W_t, hidden_t, b2)
    return logits_t.T


def kernel(context, emb_table, W, b):
    ctx3 = (
        context.reshape(_NW, _BW, CTX)
        .transpose(0, 2, 1)
        .reshape(_NW, _NCHUNK, _IDX_CHUNK)
    )
    hidden_t = _hidden_sc(ctx3, emb_table.T)
    return _projection(hidden_t, W.T, b.reshape(1, VOCAB))


# speedup vs baseline: 3.3435x; 1.4569x over previous
import functools

import jax
import jax.numpy as jnp
from jax import lax
from jax.experimental import pallas as pl
from jax.experimental.pallas import tpu as pltpu
from jax.experimental.pallas import tpu_sc as plsc

VOCAB = 100000
EMBED = 64
BATCH = 1024
CTX = 20

_NC = 2
_NS = 16
_NW = _NC * _NS
_LANES = 16
_BW = BATCH // _NW
_ROWS = _BW * CTX


def _row_copies(table_hbm, rows_v, idx_v, sem, i):
    r0 = i * CTX
    ids0 = idx_v[pl.ds(r0, _LANES)]
    ids1 = idx_v[pl.ds(r0 + _LANES, _LANES)]
    copies = []
    for t in range(CTX):
        v = ids0[t] if t < _LANES else ids1[t - _LANES]
        copies.append(
            pltpu.make_async_copy(
                table_hbm.at[pl.ds(v, 1)], rows_v.at[pl.ds(r0 + t, 1)], sem
            )
        )
    return copies


def _hidden_body(ctx_hbm, table_hbm, out_hbm, idx_v, rows_v, hid_v, sem):
    wid = lax.axis_index("s") * _NC + lax.axis_index("c")
    base = wid * _BW

    pltpu.sync_copy(
        ctx_hbm.at[pl.ds(base * CTX, _ROWS)], idx_v.at[pl.ds(0, _ROWS)]
    )

    def fire(i, _):
        for cp in _row_copies(table_hbm, rows_v, idx_v, sem, i):
            cp.start()
        return _

    lax.fori_loop(0, _BW, fire, None)

    def body(i, _):
        for cp in _row_copies(table_hbm, rows_v, idx_v, sem, i):
            cp.wait()
        r0 = i * CTX
        for c in range(EMBED // _LANES):
            sl = pl.ds(c * _LANES, _LANES)
            acc = rows_v[r0, sl]
            for t in range(1, CTX):
                acc = acc + rows_v[r0 + t, sl]
            hid_v[i, sl] = acc * (1.0 / CTX)
        return _

    lax.fori_loop(0, _BW, body, None)
    pltpu.sync_copy(hid_v, out_hbm.at[pl.ds(base, _BW)])


@functools.partial(
    pl.kernel,
    mesh=plsc.VectorSubcoreMesh(core_axis_name="c", subcore_axis_name="s"),
    out_type=jax.ShapeDtypeStruct((BATCH, EMBED), jnp.float32),
    scratch_types=[
        pltpu.VMEM((_ROWS + _LANES,), jnp.int32),
        pltpu.VMEM((_ROWS, EMBED), jnp.float32),
        pltpu.VMEM((_BW, EMBED), jnp.float32),
        pltpu.SemaphoreType.DMA,
    ],
)
def _hidden_sc(ctx_hbm, table_hbm, out_hbm, idx_v, rows_v, hid_v, sem):
    _hidden_body(ctx_hbm, table_hbm, out_hbm, idx_v, rows_v, hid_v, sem)


_VBLK = 4096


def _proj_body(wt_ref, h_ref, b_ref, o_ref):
    wt_aug = jnp.concatenate([wt_ref[...], b_ref[...]], axis=0)
    h_aug = jnp.concatenate(
        [h_ref[...], jnp.ones((BATCH, 1), jnp.float32)], axis=1
    )
    o_ref[...] = lax.dot_general(
        wt_aug,
        h_aug,
        (((0,), (1,)), ((), ())),
        preferred_element_type=jnp.float32,
    )


def _projection(hidden, W_t, b2):
    logits_t = pl.pallas_call(
        _proj_body,
        grid=(pl.cdiv(VOCAB, _VBLK),),
        in_specs=[
            pl.BlockSpec((EMBED, _VBLK), lambda j: (0, j)),
            pl.BlockSpec((BATCH, EMBED), lambda j: (0, 0)),
            pl.BlockSpec((1, _VBLK), lambda j: (0, j)),
        ],
        out_specs=pl.BlockSpec((_VBLK, BATCH), lambda j: (j, 0)),
        out_shape=jax.ShapeDtypeStruct((VOCAB, BATCH), jnp.float32),
    )(W_t, hidden, b2)
    return logits_t.T


def kernel(context, emb_table, W, b):
    hidden = _hidden_sc(context.reshape(BATCH * CTX), emb_table)
    return _projection(hidden, W.T, b.reshape(1, VOCAB))
